# baseline (speedup 1.0000x reference)
import functools

import jax
import jax.numpy as jnp
from jax import lax
from jax.experimental import pallas as pl
from jax.experimental.pallas import tpu as pltpu
from jax.experimental.pallas import tpu_sc as plsc

H = 1280
I = 896
E = 64
K = 8
S = 2048
ISH = 2 * I
A = S * K

BM = 128
NT = A // BM + E
P = NT * BM
TB = 256

_NC, _NS = 2, 16
_NW = _NC * _NS
_DCH = 32
_DPW = A // _NW
_TPW = S // _NW
_CT = 4


def _router_body(x_ref, gw_ref, w_ref, i_ref, c_ref):
    x = x_ref[...]
    logits = lax.dot_general(x, gw_ref[...], (((1,), (1,)), ((), ())),
                             preferred_element_type=jnp.float32)
    m = jnp.max(logits, axis=1, keepdims=True)
    ex = jnp.exp(logits - m)
    sm = ex / jnp.sum(ex, axis=1, keepdims=True)
    cols = lax.broadcasted_iota(jnp.int32, (TB, E), 1)
    work = sm
    ws, idxs = [], []
    hist = jnp.zeros((TB, E), jnp.int32)
    for _ in range(K):
        mx = jnp.max(work, axis=1, keepdims=True)
        am = jnp.min(jnp.where(work == mx, cols, E), axis=1, keepdims=True)
        ws.append(mx)
        idxs.append(am)
        hist = hist + jnp.where(cols == am, 1, 0)
        work = jnp.where(cols == am, -1.0, work)
    w_ref[...] = jnp.concatenate(ws, axis=1)
    i_ref[...] = jnp.concatenate(idxs, axis=1)
    c_ref[0, 0, :] = jnp.sum(hist, axis=0)


def _gemm_body(te_ref, xb_ref, at_ref, x_ref, gp_ref, up_ref, dp_ref, wp_ref,
               y_ref):
    t = pl.program_id(0)

    @pl.when(t < at_ref[0])
    def _():
        x = x_ref[...]
        g = jnp.dot(x, gp_ref[0], preferred_element_type=jnp.float32)
        u = jnp.dot(x, up_ref[0], preferred_element_type=jnp.float32)
        h = g * lax.logistic(g) * u
        y = jnp.dot(h, dp_ref[0], preferred_element_type=jnp.float32)
        y_ref[...] = y * wp_ref[...]


def _sh1_body(x_ref, g_ref, u_ref, h_ref):
    x = x_ref[...]
    g = jnp.dot(x, g_ref[...], preferred_element_type=jnp.float32)
    u = jnp.dot(x, u_ref[...], preferred_element_type=jnp.float32)
    h_ref[...] = g * lax.logistic(g) * u


def _sh2_body(h_ref, d_ref, o_ref):
    o_ref[...] = jnp.dot(h_ref[...], d_ref[...],
                         preferred_element_type=jnp.float32)


def _dispatch_body(x_hbm, src_hbm, pp_hbm, ws_hbm, out_hbm, wo_hbm,
                   idx0, idx1, pp0, pp1, wv0, wv1, buf0, buf1,
                   gs0, gs1, ss0, ss1):
    wid = lax.axis_index("s") * _NC + lax.axis_index("c")
    base0 = wid * _DPW
    idxb, ppb, bufb = (idx0, idx1), (pp0, pp1), (buf0, buf1)
    wvb = (wv0, wv1)
    gsb, ssb = (gs0, gs1), (ss0, ss1)

    def outer(i, c):
        for b in range(2):
            base = base0 + (i * 2 + b) * _DCH
            pltpu.sync_copy(src_hbm.at[pl.ds(base, _DCH)], idxb[b])
            pltpu.async_copy(x_hbm.at[idxb[b]], bufb[b], gsb[b])
        for b in range(2):
            base = base0 + (i * 2 + b) * _DCH
            pltpu.make_async_copy(x_hbm.at[idxb[b]], bufb[b], gsb[b]).wait()
            pltpu.sync_copy(pp_hbm.at[pl.ds(base, _DCH)], ppb[b])
            pltpu.async_copy(bufb[b], out_hbm.at[ppb[b]], ssb[b])
            pltpu.sync_copy(ws_hbm.at[pl.ds(base, _DCH)], wvb[b])
            pltpu.sync_copy(wvb[b], wo_hbm.at[ppb[b]])
        for b in range(2):
            pltpu.make_async_copy(bufb[b], out_hbm.at[ppb[b]], ssb[b]).wait()
        return c

    lax.fori_loop(0, _DPW // (2 * _DCH), outer, 0)


def _combine_body(y_hbm, p_hbm, sh_hbm, out_hbm,
                  idx_a, rows0, rows1, sh0, sh1, acc0, acc1,
                  gsem0, gsem1, hsem0, hsem1, osem0, osem1):
    wid = lax.axis_index("s") * _NC + lax.axis_index("c")
    rowsb, shb, accb = (rows0, rows1), (sh0, sh1), (acc0, acc1)
    gsemb, hsemb, osemb = (gsem0, gsem1), (hsem0, hsem1), (osem0, osem1)
    nch = _TPW // _CT
    tok00 = wid * _TPW

    def gth(j):
        return pltpu.make_async_copy(
            y_hbm.at[idx_a.at[pl.ds(j * _CT * K, _CT * K)]], rowsb[j % 2],
            gsemb[j % 2])

    def shload(j):
        return pltpu.make_async_copy(
            sh_hbm.at[pl.ds(tok00 + j * _CT, _CT)], shb[j % 2],
            hsemb[j % 2])

    def ostore(j):
        return pltpu.make_async_copy(
            accb[j % 2], out_hbm.at[pl.ds(tok00 + j * _CT, _CT)],
            osemb[j % 2])

    pltpu.sync_copy(p_hbm.at[pl.ds(tok00 * K, _TPW * K)], idx_a)
    gth(0).start()
    shload(0).start()
    for j in range(nch):
        b = j % 2
        if j + 1 < nch:
            gth(j + 1).start()
            shload(j + 1).start()
        if j >= 2:
            ostore(j - 2).wait()
        gth(j).wait()
        shload(j).wait()
        for t in range(_CT):
            def vec(v, c2, t=t):
                o = v * 16
                a = shb[b][t, pl.ds(o, 16)]
                for k in range(K):
                    a = a + rowsb[b][t * K + k, pl.ds(o, 16)]
                accb[b][t, pl.ds(o, 16)] = a
                return c2
            lax.fori_loop(0, H // 16, vec, 0)
        ostore(j).start()
    ostore(nch - 2).wait()
    ostore(nch - 1).wait()


def _build(interpret=False):
    router = pl.pallas_call(
        _router_body,
        grid=(S // TB,),
        in_specs=[pl.BlockSpec((TB, H), lambda i: (i, 0)),
                  pl.BlockSpec((E, H), lambda i: (0, 0))],
        out_specs=[pl.BlockSpec((TB, K), lambda i: (i, 0)),
                   pl.BlockSpec((TB, K), lambda i: (i, 0)),
                   pl.BlockSpec((1, 1, E), lambda i: (i, 0, 0))],
        out_shape=[jax.ShapeDtypeStruct((S, K), jnp.float32),
                   jax.ShapeDtypeStruct((S, K), jnp.int32),
                   jax.ShapeDtypeStruct((S // TB, 1, E), jnp.int32)],
        interpret=interpret,
    )

    gemm = pl.pallas_call(
        _gemm_body,
        grid_spec=pltpu.PrefetchScalarGridSpec(
            num_scalar_prefetch=3,
            grid=(NT,),
            in_specs=[
                pl.BlockSpec((BM, H), lambda t, te, xb, at: (xb[t], 0)),
                pl.BlockSpec((1, H, I), lambda t, te, xb, at: (te[t], 0, 0)),
                pl.BlockSpec((1, H, I), lambda t, te, xb, at: (te[t], 0, 0)),
                pl.BlockSpec((1, I, H), lambda t, te, xb, at: (te[t], 0, 0)),
                pl.BlockSpec((BM, 1), lambda t, te, xb, at: (xb[t], 0)),
            ],
            out_specs=pl.BlockSpec((BM, H), lambda t, te, xb, at: (xb[t], 0)),
        ),
        out_shape=jax.ShapeDtypeStruct((P, H), jnp.float32),
        interpret=interpret,
    )

    sh1 = pl.pallas_call(
        _sh1_body,
        grid=(S // TB,),
        in_specs=[pl.BlockSpec((TB, H), lambda i: (i, 0)),
                  pl.BlockSpec((H, ISH), lambda i: (0, 0)),
                  pl.BlockSpec((H, ISH), lambda i: (0, 0))],
        out_specs=pl.BlockSpec((TB, ISH), lambda i: (i, 0)),
        out_shape=jax.ShapeDtypeStruct((S, ISH), jnp.float32),
        interpret=interpret,
    )

    sh2 = pl.pallas_call(
        _sh2_body,
        grid=(S // TB,),
        in_specs=[pl.BlockSpec((TB, ISH), lambda i: (i, 0)),
                  pl.BlockSpec((ISH, H), lambda i: (0, 0))],
        out_specs=pl.BlockSpec((TB, H), lambda i: (i, 0)),
        out_shape=jax.ShapeDtypeStruct((S, H), jnp.float32),
        interpret=interpret,
    )

    return router, gemm, sh1, sh2


@functools.lru_cache(maxsize=None)
def _build_sc():
    mesh = plsc.VectorSubcoreMesh(core_axis_name="c", subcore_axis_name="s",
                                  num_cores=_NC, num_subcores=_NS)

    dispatch = pl.kernel(
        _dispatch_body,
        out_type=[jax.ShapeDtypeStruct((P, H), jnp.float32),
                  jax.ShapeDtypeStruct((P,), jnp.float32)],
        mesh=mesh,
        scratch_types=[pltpu.VMEM((_DCH,), jnp.int32),
                       pltpu.VMEM((_DCH,), jnp.int32),
                       pltpu.VMEM((_DCH,), jnp.int32),
                       pltpu.VMEM((_DCH,), jnp.int32),
                       pltpu.VMEM((_DCH,), jnp.float32),
                       pltpu.VMEM((_DCH,), jnp.float32),
                       pltpu.VMEM((_DCH, H), jnp.float32),
                       pltpu.VMEM((_DCH, H), jnp.float32),
                       pltpu.SemaphoreType.DMA,
                       pltpu.SemaphoreType.DMA,
                       pltpu.SemaphoreType.DMA,
                       pltpu.SemaphoreType.DMA],
    )

    combine = pl.kernel(
        _combine_body,
        out_type=jax.ShapeDtypeStruct((S, H), jnp.float32),
        mesh=mesh,
        scratch_types=[pltpu.VMEM((_TPW * K,), jnp.int32),
                       pltpu.VMEM((_CT * K, H), jnp.float32),
                       pltpu.VMEM((_CT * K, H), jnp.float32),
                       pltpu.VMEM((_CT, H), jnp.float32),
                       pltpu.VMEM((_CT, H), jnp.float32),
                       pltpu.VMEM((_CT, H), jnp.float32),
                       pltpu.VMEM((_CT, H), jnp.float32),
                       pltpu.SemaphoreType.DMA,
                       pltpu.SemaphoreType.DMA,
                       pltpu.SemaphoreType.DMA,
                       pltpu.SemaphoreType.DMA,
                       pltpu.SemaphoreType.DMA,
                       pltpu.SemaphoreType.DMA],
    )

    return dispatch, combine


_ROUTER, _GEMM, _SH1, _SH2 = _build()


def _dispatch_plan(topk_idx, counts):
    e_flat = topk_idx.reshape(-1)
    iota = jnp.arange(A, dtype=jnp.int32)
    e_sorted, perm = lax.sort_key_val(e_flat, iota)
    group_start = (jnp.cumsum(counts) - counts).astype(jnp.int32)
    tiles_per = (counts + BM - 1) // BM
    tile_cum = jnp.cumsum(tiles_per).astype(jnp.int32)
    at = tile_cum[-1]
    pad_off = ((tile_cum - tiles_per) * BM).astype(jnp.int32)
    g_tab = pad_off - group_start
    padded_pos = g_tab[e_sorted] + iota
    src_tok = lax.shift_right_logical(perm, 3)
    _, p_flat = lax.sort_key_val(perm, padded_pos)
    t_ar = jnp.arange(NT, dtype=jnp.int32)
    te_raw = jnp.sum((tile_cum[None, :] <= t_ar[:, None]).astype(jnp.int32),
                     axis=1)
    te_last = jnp.sum((tile_cum <= at - 1).astype(jnp.int32))
    tile_expert = jnp.where(t_ar < at, jnp.minimum(te_raw, E - 1), te_last)
    xblk = jnp.minimum(t_ar, at - 1).astype(jnp.int32)
    at_arr = jnp.full((1,), at, jnp.int32)
    return tile_expert, xblk, at_arr, src_tok, padded_pos, p_flat, perm


def kernel(hidden_states, gate_weight, expert_gate_proj, expert_up_proj,
           expert_down_proj, shared_gate_proj, shared_up_proj,
           shared_down_proj):
    orig_shape = hidden_states.shape
    x = hidden_states.reshape(S, H)
    dispatch, combine = _build_sc()
    topk_w, topk_idx, hist = _ROUTER(x, gate_weight)
    counts = hist.reshape(S // TB, E).sum(axis=0)
    tile_expert, xblk, at_arr, src_tok, padded_pos, p_flat, perm = (
        _dispatch_plan(topk_idx, counts))
    w_sorted = topk_w.reshape(-1)[perm]
    x_sorted, w_pad = dispatch(x, src_tok, padded_pos, w_sorted)
    y = _GEMM(tile_expert, xblk, at_arr, x_sorted, expert_gate_proj,
              expert_up_proj, expert_down_proj, w_pad.reshape(P, 1))
    h_sh = _SH1(x, shared_gate_proj, shared_up_proj)
    sh_out = _SH2(h_sh, shared_down_proj)
    out = combine(y, p_flat, sh_out)
    return out.reshape(orig_shape)


# speedup vs baseline: 1.0232x; 1.0232x over previous
import functools

import jax
import jax.numpy as jnp
from jax import lax
from jax.experimental import pallas as pl
from jax.experimental.pallas import tpu as pltpu
from jax.experimental.pallas import tpu_sc as plsc

H = 1280
I = 896
E = 64
K = 8
S = 2048
ISH = 2 * I
A = S * K

BM = 128
NT = A // BM + E
P = NT * BM
TB = 256

_NC, _NS = 2, 16
_NW = _NC * _NS
_DCH = 32
_DPW = A // _NW
_TPW = S // _NW
_CT = 4


def _router_body(x_ref, gw_ref, w_ref, i_ref, c_ref):
    x = x_ref[...]
    logits = lax.dot_general(x, gw_ref[...], (((1,), (1,)), ((), ())),
                             preferred_element_type=jnp.float32)
    m = jnp.max(logits, axis=1, keepdims=True)
    ex = jnp.exp(logits - m)
    sm = ex / jnp.sum(ex, axis=1, keepdims=True)
    cols = lax.broadcasted_iota(jnp.int32, (TB, E), 1)
    work = sm
    ws, idxs = [], []
    hist = jnp.zeros((TB, E), jnp.int32)
    for _ in range(K):
        mx = jnp.max(work, axis=1, keepdims=True)
        am = jnp.min(jnp.where(work == mx, cols, E), axis=1, keepdims=True)
        ws.append(mx)
        idxs.append(am)
        hist = hist + jnp.where(cols == am, 1, 0)
        work = jnp.where(cols == am, -1.0, work)
    w_ref[...] = jnp.concatenate(ws, axis=1)
    i_ref[...] = jnp.concatenate(idxs, axis=1)
    c_ref[0, 0, :] = jnp.sum(hist, axis=0)


def _gemm_body(te_ref, xb_ref, at_ref, x_ref, gp_ref, up_ref, dp_ref, wp_ref,
               y_ref):
    t = pl.program_id(0)

    @pl.when(t < at_ref[0])
    def _():
        x = x_ref[...]
        g = jnp.dot(x, gp_ref[0], preferred_element_type=jnp.float32)
        u = jnp.dot(x, up_ref[0], preferred_element_type=jnp.float32)
        h = g * lax.logistic(g) * u
        y = jnp.dot(h, dp_ref[0], preferred_element_type=jnp.float32)
        y_ref[...] = y * wp_ref[...]


def _sh1_body(x_ref, g_ref, u_ref, h_ref):
    x = x_ref[...]
    g = jnp.dot(x, g_ref[...], preferred_element_type=jnp.float32)
    u = jnp.dot(x, u_ref[...], preferred_element_type=jnp.float32)
    h_ref[...] = g * lax.logistic(g) * u


def _sh2_body(h_ref, d_ref, o_ref):
    o_ref[...] = jnp.dot(h_ref[...], d_ref[...],
                         preferred_element_type=jnp.float32)


def _dispatch_body(x_hbm, src_hbm, pp_hbm, out_hbm,
                   idx0, idx1, pp0, pp1, buf0, buf1,
                   gs0, gs1, ss0, ss1):
    wid = lax.axis_index("s") * _NC + lax.axis_index("c")
    base0 = wid * _DPW
    idxb, ppb, bufb = (idx0, idx1), (pp0, pp1), (buf0, buf1)
    gsb, ssb = (gs0, gs1), (ss0, ss1)

    def outer(i, c):
        for b in range(2):
            base = base0 + (i * 2 + b) * _DCH
            pltpu.sync_copy(src_hbm.at[pl.ds(base, _DCH)], idxb[b])
            pltpu.async_copy(x_hbm.at[idxb[b]], bufb[b], gsb[b])
        for b in range(2):
            base = base0 + (i * 2 + b) * _DCH
            pltpu.make_async_copy(x_hbm.at[idxb[b]], bufb[b], gsb[b]).wait()
            pltpu.sync_copy(pp_hbm.at[pl.ds(base, _DCH)], ppb[b])
            pltpu.async_copy(bufb[b], out_hbm.at[ppb[b]], ssb[b])
        for b in range(2):
            pltpu.make_async_copy(bufb[b], out_hbm.at[ppb[b]], ssb[b]).wait()
        return c

    lax.fori_loop(0, _DPW // (2 * _DCH), outer, 0)


def _combine_body(y_hbm, p_hbm, sh_hbm, out_hbm,
                  idx_a, rows0, rows1, sh0, sh1, acc0, acc1,
                  gsem0, gsem1, hsem0, hsem1, osem0, osem1):
    wid = lax.axis_index("s") * _NC + lax.axis_index("c")
    rowsb, shb, accb = (rows0, rows1), (sh0, sh1), (acc0, acc1)
    gsemb, hsemb, osemb = (gsem0, gsem1), (hsem0, hsem1), (osem0, osem1)
    nch = _TPW // _CT
    tok00 = wid * _TPW

    def gth(j):
        return pltpu.make_async_copy(
            y_hbm.at[idx_a.at[pl.ds(j * _CT * K, _CT * K)]], rowsb[j % 2],
            gsemb[j % 2])

    def shload(j):
        return pltpu.make_async_copy(
            sh_hbm.at[pl.ds(tok00 + j * _CT, _CT)], shb[j % 2],
            hsemb[j % 2])

    def ostore(j):
        return pltpu.make_async_copy(
            accb[j % 2], out_hbm.at[pl.ds(tok00 + j * _CT, _CT)],
            osemb[j % 2])

    pltpu.sync_copy(p_hbm.at[pl.ds(tok00 * K, _TPW * K)], idx_a)
    gth(0).start()
    shload(0).start()
    for j in range(nch):
        b = j % 2
        if j + 1 < nch:
            gth(j + 1).start()
            shload(j + 1).start()
        if j >= 2:
            ostore(j - 2).wait()
        gth(j).wait()
        shload(j).wait()
        for t in range(_CT):
            def vec(v, c2, t=t):
                o = v * 16
                a = shb[b][t, pl.ds(o, 16)]
                for k in range(K):
                    a = a + rowsb[b][t * K + k, pl.ds(o, 16)]
                accb[b][t, pl.ds(o, 16)] = a
                return c2
            lax.fori_loop(0, H // 16, vec, 0)
        ostore(j).start()
    ostore(nch - 2).wait()
    ostore(nch - 1).wait()


def _build(interpret=False):
    router = pl.pallas_call(
        _router_body,
        grid=(S // TB,),
        in_specs=[pl.BlockSpec((TB, H), lambda i: (i, 0)),
                  pl.BlockSpec((E, H), lambda i: (0, 0))],
        out_specs=[pl.BlockSpec((TB, K), lambda i: (i, 0)),
                   pl.BlockSpec((TB, K), lambda i: (i, 0)),
                   pl.BlockSpec((1, 1, E), lambda i: (i, 0, 0))],
        out_shape=[jax.ShapeDtypeStruct((S, K), jnp.float32),
                   jax.ShapeDtypeStruct((S, K), jnp.int32),
                   jax.ShapeDtypeStruct((S // TB, 1, E), jnp.int32)],
        interpret=interpret,
    )

    gemm = pl.pallas_call(
        _gemm_body,
        grid_spec=pltpu.PrefetchScalarGridSpec(
            num_scalar_prefetch=3,
            grid=(NT,),
            in_specs=[
                pl.BlockSpec((BM, H), lambda t, te, xb, at: (xb[t], 0)),
                pl.BlockSpec((1, H, I), lambda t, te, xb, at: (te[t], 0, 0)),
                pl.BlockSpec((1, H, I), lambda t, te, xb, at: (te[t], 0, 0)),
                pl.BlockSpec((1, I, H), lambda t, te, xb, at: (te[t], 0, 0)),
                pl.BlockSpec((BM, 1), lambda t, te, xb, at: (xb[t], 0)),
            ],
            out_specs=pl.BlockSpec((BM, H), lambda t, te, xb, at: (xb[t], 0)),
        ),
        out_shape=jax.ShapeDtypeStruct((P, H), jnp.float32),
        interpret=interpret,
    )

    sh1 = pl.pallas_call(
        _sh1_body,
        grid=(S // TB,),
        in_specs=[pl.BlockSpec((TB, H), lambda i: (i, 0)),
                  pl.BlockSpec((H, ISH), lambda i: (0, 0)),
                  pl.BlockSpec((H, ISH), lambda i: (0, 0))],
        out_specs=pl.BlockSpec((TB, ISH), lambda i: (i, 0)),
        out_shape=jax.ShapeDtypeStruct((S, ISH), jnp.float32),
        interpret=interpret,
    )

    sh2 = pl.pallas_call(
        _sh2_body,
        grid=(S // TB,),
        in_specs=[pl.BlockSpec((TB, ISH), lambda i: (i, 0)),
                  pl.BlockSpec((ISH, H), lambda i: (0, 0))],
        out_specs=pl.BlockSpec((TB, H), lambda i: (i, 0)),
        out_shape=jax.ShapeDtypeStruct((S, H), jnp.float32),
        interpret=interpret,
    )

    return router, gemm, sh1, sh2


@functools.lru_cache(maxsize=None)
def _build_sc():
    mesh = plsc.VectorSubcoreMesh(core_axis_name="c", subcore_axis_name="s",
                                  num_cores=_NC, num_subcores=_NS)

    dispatch = pl.kernel(
        _dispatch_body,
        out_type=jax.ShapeDtypeStruct((P, H), jnp.float32),
        mesh=mesh,
        scratch_types=[pltpu.VMEM((_DCH,), jnp.int32),
                       pltpu.VMEM((_DCH,), jnp.int32),
                       pltpu.VMEM((_DCH,), jnp.int32),
                       pltpu.VMEM((_DCH,), jnp.int32),
                       pltpu.VMEM((_DCH, H), jnp.float32),
                       pltpu.VMEM((_DCH, H), jnp.float32),
                       pltpu.SemaphoreType.DMA,
                       pltpu.SemaphoreType.DMA,
                       pltpu.SemaphoreType.DMA,
                       pltpu.SemaphoreType.DMA],
    )

    combine = pl.kernel(
        _combine_body,
        out_type=jax.ShapeDtypeStruct((S, H), jnp.float32),
        mesh=mesh,
        scratch_types=[pltpu.VMEM((_TPW * K,), jnp.int32),
                       pltpu.VMEM((_CT * K, H), jnp.float32),
                       pltpu.VMEM((_CT * K, H), jnp.float32),
                       pltpu.VMEM((_CT, H), jnp.float32),
                       pltpu.VMEM((_CT, H), jnp.float32),
                       pltpu.VMEM((_CT, H), jnp.float32),
                       pltpu.VMEM((_CT, H), jnp.float32),
                       pltpu.SemaphoreType.DMA,
                       pltpu.SemaphoreType.DMA,
                       pltpu.SemaphoreType.DMA,
                       pltpu.SemaphoreType.DMA,
                       pltpu.SemaphoreType.DMA,
                       pltpu.SemaphoreType.DMA],
    )

    return dispatch, combine


_ROUTER, _GEMM, _SH1, _SH2 = _build()


def _dispatch_plan(topk_idx, counts):
    e_flat = topk_idx.reshape(-1)
    iota = jnp.arange(A, dtype=jnp.int32)
    e_sorted, perm = lax.sort_key_val(e_flat, iota)
    group_start = (jnp.cumsum(counts) - counts).astype(jnp.int32)
    tiles_per = (counts + BM - 1) // BM
    tile_cum = jnp.cumsum(tiles_per).astype(jnp.int32)
    at = tile_cum[-1]
    pad_off = ((tile_cum - tiles_per) * BM).astype(jnp.int32)
    g_tab = pad_off - group_start
    padded_pos = g_tab[e_sorted] + iota
    src_tok = lax.shift_right_logical(perm, 3)
    _, p_flat = lax.sort_key_val(perm, padded_pos)
    t_ar = jnp.arange(NT, dtype=jnp.int32)
    te_raw = jnp.sum((tile_cum[None, :] <= t_ar[:, None]).astype(jnp.int32),
                     axis=1)
    te_last = jnp.sum((tile_cum <= at - 1).astype(jnp.int32))
    tile_expert = jnp.where(t_ar < at, jnp.minimum(te_raw, E - 1), te_last)
    xblk = jnp.minimum(t_ar, at - 1).astype(jnp.int32)
    at_arr = jnp.full((1,), at, jnp.int32)
    return tile_expert, xblk, at_arr, src_tok, padded_pos, p_flat, perm


def kernel(hidden_states, gate_weight, expert_gate_proj, expert_up_proj,
           expert_down_proj, shared_gate_proj, shared_up_proj,
           shared_down_proj):
    orig_shape = hidden_states.shape
    x = hidden_states.reshape(S, H)
    dispatch, combine = _build_sc()
    topk_w, topk_idx, hist = _ROUTER(x, gate_weight)
    counts = hist.reshape(S // TB, E).sum(axis=0)
    tile_expert, xblk, at_arr, src_tok, padded_pos, p_flat, perm = (
        _dispatch_plan(topk_idx, counts))
    w_sorted = topk_w.reshape(-1)[perm]
    w_pad = jnp.zeros((P,), jnp.float32).at[padded_pos].set(w_sorted)
    x_sorted = dispatch(x, src_tok, padded_pos)
    y = _GEMM(tile_expert, xblk, at_arr, x_sorted, expert_gate_proj,
              expert_up_proj, expert_down_proj, w_pad.reshape(P, 1))
    h_sh = _SH1(x, shared_gate_proj, shared_up_proj)
    sh_out = _SH2(h_sh, shared_down_proj)
    out = combine(y, p_flat, sh_out)
    return out.reshape(orig_shape)
